# baseline (speedup 1.0000x reference)
import functools

import jax
import jax.numpy as jnp
from jax.experimental import pallas as pl
from jax.experimental.pallas import tpu as pltpu

_GBLK = 512
_CBLK = 2048


def _gather_body(t_ref, x_ref, v_ref):
    i = pl.program_id(0)
    t = t_ref[i]
    loc = t - (t // _GBLK) * _GBLK
    lane = jax.lax.broadcasted_iota(jnp.int32, (1, 1, _GBLK), 2)
    sel = jnp.where(lane == loc, x_ref[...], 0.0)
    v_ref[...] = jnp.sum(sel, axis=(1, 2), keepdims=True)


def _count_body(x_ref, v_ref, t_ref, out1_ref, out5_ref, acc_ref, *, nsteps, n, bsz):
    j = pl.program_id(0)

    @pl.when(j == 0)
    def _init():
        acc_ref[...] = jnp.zeros_like(acc_ref)

    x = x_ref[...]
    v = v_ref[...]
    t = t_ref[...]
    col = j * _CBLK + jax.lax.broadcasted_iota(jnp.int32, x.shape, 1)
    beat = (col < n) & ((x > v) | ((x == v) & (col < t)))
    acc_ref[...] += jnp.sum(beat.astype(jnp.int32), axis=1, keepdims=True)

    @pl.when(j == nsteps - 1)
    def _fin():
        rank = acc_ref[...]
        scale = 100.0 / bsz
        out1_ref[...] = jnp.sum((rank < 1).astype(jnp.float32),
                                axis=(0, 1), keepdims=True) * scale
        out5_ref[...] = jnp.sum((rank < 5).astype(jnp.float32),
                                axis=(0, 1), keepdims=True) * scale


def kernel(preds, targets):
    bsz, n = preds.shape
    t32 = targets.astype(jnp.int32)

    v = pl.pallas_call(
        _gather_body,
        grid_spec=pltpu.PrefetchScalarGridSpec(
            num_scalar_prefetch=1,
            grid=(bsz,),
            in_specs=[pl.BlockSpec((1, 1, _GBLK),
                                   lambda i, t: (i, 0, t[i] // _GBLK))],
            out_specs=pl.BlockSpec((1, 1, 1), lambda i, t: (i, 0, 0)),
        ),
        out_shape=jax.ShapeDtypeStruct((bsz, 1, 1), jnp.float32),
    )(t32, preds.reshape(bsz, 1, n))
    v = v.reshape(bsz, 1)

    nsteps = pl.cdiv(n, _CBLK)
    out1, out5 = pl.pallas_call(
        functools.partial(_count_body, nsteps=nsteps, n=n, bsz=bsz),
        grid=(nsteps,),
        in_specs=[
            pl.BlockSpec((bsz, _CBLK), lambda j: (0, j)),
            pl.BlockSpec((bsz, 1), lambda j: (0, 0)),
            pl.BlockSpec((bsz, 1), lambda j: (0, 0)),
        ],
        out_specs=[
            pl.BlockSpec((1, 1), lambda j: (0, 0)),
            pl.BlockSpec((1, 1), lambda j: (0, 0)),
        ],
        out_shape=[jax.ShapeDtypeStruct((1, 1), jnp.float32)] * 2,
        scratch_shapes=[pltpu.VMEM((bsz, 1), jnp.int32)],
    )(preds, v, t32.reshape(bsz, 1))

    return (out1.reshape(1), out5.reshape(1))


# speedup vs baseline: 1.4382x; 1.4382x over previous
import functools

import jax
import jax.numpy as jnp
from jax import lax
from jax.experimental import pallas as pl
from jax.experimental.pallas import tpu as pltpu
from jax.experimental.pallas import tpu_sc as plsc

_CBLK = 2048
_L = 16


def _count_body(x_ref, v_ref, t_ref, out1_ref, out5_ref, acc_ref, *, nsteps, n, bsz):
    j = pl.program_id(0)

    @pl.when(j == 0)
    def _init():
        acc_ref[...] = jnp.zeros_like(acc_ref)

    x = x_ref[...]
    v = v_ref[...]
    t = t_ref[...]
    col = j * _CBLK + jax.lax.broadcasted_iota(jnp.int32, x.shape, 1)
    beat = (col < n) & ((x > v) | ((x == v) & (col < t)))
    acc_ref[...] += jnp.sum(beat.astype(jnp.int32), axis=1, keepdims=True)

    @pl.when(j == nsteps - 1)
    def _fin():
        rank = acc_ref[...]
        scale = 100.0 / bsz
        out1_ref[...] = jnp.sum((rank < 1).astype(jnp.float32),
                                axis=(0, 1), keepdims=True) * scale
        out5_ref[...] = jnp.sum((rank < 5).astype(jnp.float32),
                                axis=(0, 1), keepdims=True) * scale


def kernel(preds, targets):
    bsz, n = preds.shape
    t32 = targets.astype(jnp.int32)

    info = plsc.get_sparse_core_info()
    nc = info.num_cores
    nworkers = bsz // _L

    @functools.partial(
        pl.kernel,
        out_type=jax.ShapeDtypeStruct((bsz,), jnp.float32),
        mesh=plsc.VectorSubcoreMesh(core_axis_name="c", subcore_axis_name="s"),
        scratch_types=[
            pltpu.VMEM((_L,), jnp.int32),
            pltpu.VMEM((_L,), jnp.float32),
            pltpu.SemaphoreType.DMA,
        ],
    )
    def _sc_gather(pf_ref, t_ref, v_ref, idx_v, val_v, sem):
        wid = lax.axis_index("s") * nc + lax.axis_index("c")

        @pl.when(wid < nworkers)
        def _():
            base = wid * _L
            pltpu.sync_copy(t_ref.at[pl.ds(base, _L)], idx_v)
            rows = base + lax.iota(jnp.int32, _L)
            idx_v[...] = rows * n + idx_v[...]
            pltpu.async_copy(pf_ref.at[idx_v], val_v, sem).wait()
            pltpu.sync_copy(val_v, v_ref.at[pl.ds(base, _L)])

    v = _sc_gather(preds.reshape(bsz * n), t32).reshape(bsz, 1)

    nsteps = pl.cdiv(n, _CBLK)
    out1, out5 = pl.pallas_call(
        functools.partial(_count_body, nsteps=nsteps, n=n, bsz=bsz),
        grid=(nsteps,),
        in_specs=[
            pl.BlockSpec((bsz, _CBLK), lambda j: (0, j)),
            pl.BlockSpec((bsz, 1), lambda j: (0, 0)),
            pl.BlockSpec((bsz, 1), lambda j: (0, 0)),
        ],
        out_specs=[
            pl.BlockSpec((1, 1), lambda j: (0, 0)),
            pl.BlockSpec((1, 1), lambda j: (0, 0)),
        ],
        out_shape=[jax.ShapeDtypeStruct((1, 1), jnp.float32)] * 2,
        scratch_shapes=[pltpu.VMEM((bsz, 1), jnp.int32)],
    )(preds, v, t32.reshape(bsz, 1))

    return (out1.reshape(1), out5.reshape(1))
